# baseline (speedup 1.0000x reference)
import functools

import jax
import jax.numpy as jnp
from jax.experimental import pallas as pl
from jax.experimental.pallas import tpu as pltpu

_C = 1.0
_MIN_NORM = 1e-15
_PROJ_EPS = 4e-3

_N = 8192
_D = 128
_DOUT = 16
_NUM_ADJS = 2

_BM = 1024
_BK = 2048


def _support_body(x_ref, w_ref, s_ref):
    x = x_ref[...]
    n = jnp.maximum(jnp.sqrt(jnp.sum(x * x, axis=-1, keepdims=True)), _MIN_NORM)
    t = jnp.clip(n, -1.0 + 1e-7, 1.0 - 1e-7)
    atanh_t = 0.5 * (jnp.log1p(t) - jnp.log1p(-t))
    xt = (atanh_t / n) * x
    s_ref[...] = jnp.dot(xt, w_ref[...], preferred_element_type=jnp.float32)


def _expmap_proj(u):
    n = jnp.maximum(jnp.sqrt(jnp.sum(u * u, axis=-1, keepdims=True)), _MIN_NORM)
    e = jnp.tanh(n) * u / n
    rn = jnp.maximum(jnp.sqrt(jnp.sum(e * e, axis=-1, keepdims=True)), _MIN_NORM)
    maxnorm = 1.0 - _PROJ_EPS
    return jnp.where(rn > maxnorm, e / rn * maxnorm, e)


def _agg_body(adj_ref, s_ref, b_ref, out_ref, acc_ref, *, nk):
    j = pl.program_id(1)

    @pl.when(j == 0)
    def _init():
        acc_ref[...] = jnp.zeros_like(acc_ref)

    a = adj_ref[...]
    s = s_ref[...]
    acc_ref[:, :_DOUT] += jnp.dot(a[0], s[:, :_DOUT],
                                  preferred_element_type=jnp.float32)
    acc_ref[:, _DOUT:] += jnp.dot(a[1], s[:, _DOUT:],
                                  preferred_element_type=jnp.float32)

    @pl.when(j == nk - 1)
    def _epilogue():
        acc = acc_ref[...]
        e0 = _expmap_proj(acc[:, :_DOUT])
        e1 = _expmap_proj(acc[:, _DOUT:])
        out_ref[...] = (e0 + e1) / 11.0 + b_ref[...]


def kernel(x, adj, adj_weight, bias):
    w2 = jnp.transpose(adj_weight, (1, 0, 2)).reshape(_D, _NUM_ADJS * _DOUT)

    s = pl.pallas_call(
        _support_body,
        out_shape=jax.ShapeDtypeStruct((_N, _NUM_ADJS * _DOUT), jnp.float32),
    )(x, w2)

    nr = _N // _BM
    nk = _N // _BK
    out = pl.pallas_call(
        functools.partial(_agg_body, nk=nk),
        grid=(nr, nk),
        in_specs=[
            pl.BlockSpec((_NUM_ADJS, _BM, _BK), lambda i, j: (0, i, j)),
            pl.BlockSpec((_BK, _NUM_ADJS * _DOUT), lambda i, j: (j, 0)),
            pl.BlockSpec((1, _DOUT), lambda i, j: (0, 0)),
        ],
        out_specs=pl.BlockSpec((_BM, _DOUT), lambda i, j: (i, 0)),
        out_shape=jax.ShapeDtypeStruct((_N, _DOUT), jnp.float32),
        scratch_shapes=[pltpu.VMEM((_BM, _NUM_ADJS * _DOUT), jnp.float32)],
        compiler_params=pltpu.CompilerParams(
            dimension_semantics=("parallel", "arbitrary"),
        ),
    )(adj, s, bias.reshape(1, _DOUT))
    return out


# speedup vs baseline: 1.1415x; 1.1415x over previous
import jax
import jax.numpy as jnp
from jax.experimental import pallas as pl
from jax.experimental.pallas import tpu as pltpu

_MIN_NORM = 1e-15
_PROJ_EPS = 4e-3

_N = 8192
_D = 128
_DOUT = 16
_NUM_ADJS = 2

_BM = 256


def _expmap_proj(u):
    n = jnp.maximum(jnp.sqrt(jnp.sum(u * u, axis=-1, keepdims=True)), _MIN_NORM)
    e = jnp.tanh(n) * u / n
    rn = jnp.maximum(jnp.sqrt(jnp.sum(e * e, axis=-1, keepdims=True)), _MIN_NORM)
    maxnorm = 1.0 - _PROJ_EPS
    return jnp.where(rn > maxnorm, e / rn * maxnorm, e)


def _body(x_ref, w_ref, b_ref, adj_ref, out_ref, s_ref):
    i = pl.program_id(0)

    @pl.when(i == 0)
    def _support():
        x = x_ref[...]
        n = jnp.maximum(jnp.sqrt(jnp.sum(x * x, axis=-1, keepdims=True)),
                        _MIN_NORM)
        t = jnp.clip(n, -1.0 + 1e-7, 1.0 - 1e-7)
        atanh_t = 0.5 * (jnp.log1p(t) - jnp.log1p(-t))
        xt = (atanh_t / n) * x
        s_ref[...] = jnp.dot(xt, w_ref[...],
                             preferred_element_type=jnp.float32)

    a = adj_ref[...]
    s = s_ref[...]
    u0 = jnp.dot(a[0], s[:, :_DOUT], preferred_element_type=jnp.float32)
    u1 = jnp.dot(a[1], s[:, _DOUT:], preferred_element_type=jnp.float32)
    out_ref[...] = (_expmap_proj(u0) + _expmap_proj(u1)) / 11.0 + b_ref[...]


def kernel(x, adj, adj_weight, bias):
    w2 = jnp.transpose(adj_weight, (1, 0, 2)).reshape(_D, _NUM_ADJS * _DOUT)

    nr = _N // _BM
    out = pl.pallas_call(
        _body,
        grid=(nr,),
        in_specs=[
            pl.BlockSpec((_N, _D), lambda i: (0, 0)),
            pl.BlockSpec((_D, _NUM_ADJS * _DOUT), lambda i: (0, 0)),
            pl.BlockSpec((1, _DOUT), lambda i: (0, 0)),
            pl.BlockSpec((_NUM_ADJS, _BM, _N), lambda i: (0, i, 0)),
        ],
        out_specs=pl.BlockSpec((_BM, _DOUT), lambda i: (i, 0)),
        out_shape=jax.ShapeDtypeStruct((_N, _DOUT), jnp.float32),
        scratch_shapes=[pltpu.VMEM((_N, _NUM_ADJS * _DOUT), jnp.float32)],
        compiler_params=pltpu.CompilerParams(
            dimension_semantics=("arbitrary",),
        ),
    )(x, w2, bias.reshape(1, _DOUT), adj)
    return out
